# baseline (speedup 1.0000x reference)
import functools

import jax
import jax.numpy as jnp
from jax import lax
from jax.experimental import pallas as pl
from jax.experimental.pallas import tpu as pltpu
from jax.experimental.pallas import tpu_sc as plsc

VOCAB = 49408
SEQ = 77
DIM = 768
BATCH = 1024
LANES = 16

NC = 2
NS = 16
NW = NC * NS
ROWS = BATCH * SEQ
RPW = ROWS // NW
CH = 32
NCH = RPW // CH
CVEC = DIM // LANES


def _body(table_hbm, idx_hbm, pos_hbm, out_hbm,
          idx_v, pos_v, buf0, buf1, sem0, sem1):
    wid = lax.axis_index("s") * NC + lax.axis_index("c")
    base = wid * RPW

    pltpu.sync_copy(idx_hbm.at[wid], idx_v)
    pltpu.sync_copy(pos_hbm, pos_v)

    pltpu.async_copy(table_hbm.at[idx_v.at[0]], buf0, sem0)
    pltpu.async_copy(table_hbm.at[idx_v.at[1]], buf1, sem1)

    def process(g, buf, sem):
        pltpu.make_async_copy(table_hbm.at[idx_v.at[g]], buf, sem).wait()

        def row_body(r, carry):
            pr = lax.rem(g * CH + r, SEQ)
            for c in range(CVEC):
                sl = pl.ds(c * LANES, LANES)
                buf[r, sl] = buf[r, sl] + pos_v[pr, sl]
            return carry

        lax.fori_loop(0, CH, row_body, 0)
        pltpu.sync_copy(buf, out_hbm.at[pl.ds(base + g * CH, CH)])
        gn = jnp.minimum(g + 2, NCH - 1)
        pltpu.async_copy(table_hbm.at[idx_v.at[gn]], buf, sem)

    def outer(t, carry):
        g0 = 2 * t
        process(g0, buf0, sem0)
        process(g0 + 1, buf1, sem1)
        return carry

    lax.fori_loop(0, NCH // 2, outer, 0)
    process(NCH - 1, buf0, sem0)

    pltpu.make_async_copy(table_hbm.at[idx_v.at[NCH - 1]], buf0, sem0).wait()
    pltpu.make_async_copy(table_hbm.at[idx_v.at[NCH - 1]], buf1, sem1).wait()


@functools.partial(jax.jit, donate_argnums=())
def _run(table, idx, pos):
    mesh = plsc.VectorSubcoreMesh(core_axis_name="c", subcore_axis_name="s")
    f = pl.kernel(
        _body,
        out_type=jax.ShapeDtypeStruct((ROWS, DIM), jnp.float32),
        mesh=mesh,
        scratch_types=[
            pltpu.VMEM((NCH, CH), jnp.int32),
            pltpu.VMEM((SEQ, DIM), jnp.float32),
            pltpu.VMEM((CH, DIM), jnp.float32),
            pltpu.VMEM((CH, DIM), jnp.float32),
            pltpu.SemaphoreType.DMA,
            pltpu.SemaphoreType.DMA,
        ],
    )
    return f(table, idx, pos)


def kernel(input, embed_table, pos_table):
    idx = jnp.reshape(input.astype(jnp.int32), (NW, NCH, CH))
    out = _run(embed_table, idx, pos_table)
    return jnp.reshape(out, (BATCH, SEQ, DIM))


# speedup vs baseline: 1.0758x; 1.0758x over previous
import functools

import jax
import jax.numpy as jnp
from jax import lax
from jax.experimental import pallas as pl
from jax.experimental.pallas import tpu as pltpu
from jax.experimental.pallas import tpu_sc as plsc

VOCAB = 49408
SEQ = 77
DIM = 768
BATCH = 1024
LANES = 16

NC = 2
NS = 16
NW = NC * NS
ROWS = BATCH * SEQ
RPW = ROWS // NW
CH = 16
NCH = RPW // CH
NBUF = 3
CVEC = DIM // LANES


def _body(table_hbm, idx_hbm, pos_hbm, out_hbm,
          idx_v, pos_v, buf0, buf1, buf2,
          gs0, gs1, gs2, ws0, ws1, ws2):
    bufs = [buf0, buf1, buf2]
    gsems = [gs0, gs1, gs2]
    wsems = [ws0, ws1, ws2]

    wid = lax.axis_index("s") * NC + lax.axis_index("c")
    base = wid * RPW

    pltpu.sync_copy(idx_hbm.at[wid], idx_v)
    pltpu.sync_copy(pos_hbm, pos_v)

    pltpu.async_copy(table_hbm.at[idx_v.at[0]], buf0, gs0)
    pltpu.async_copy(table_hbm.at[idx_v.at[1]], buf1, gs1)

    def step(g, b):
        buf, gsem, wsem = bufs[b], gsems[b], wsems[b]
        pltpu.make_async_copy(table_hbm.at[idx_v.at[g]], buf, gsem).wait()

        def row_body(r, carry):
            pr = lax.rem(g * CH + r, SEQ)
            for c in range(CVEC):
                sl = pl.ds(c * LANES, LANES)
                buf[r, sl] = buf[r, sl] + pos_v[pr, sl]
            return carry

        lax.fori_loop(0, CH, row_body, 0)

        out_ref = out_hbm.at[pl.ds(base + g * CH, CH)]
        pltpu.async_copy(buf, out_ref, wsem)

        bn = (b + 2) % NBUF
        gp = g - 1

        @pl.when(g >= 1)
        def _():
            prev_out = out_hbm.at[pl.ds(base + gp * CH, CH)]
            pltpu.make_async_copy(bufs[bn], prev_out, wsems[bn]).wait()

        @pl.when(g + 2 <= NCH - 1)
        def _():
            gn = jnp.minimum(g + 2, NCH - 1)
            pltpu.async_copy(table_hbm.at[idx_v.at[gn]], bufs[bn], gsems[bn])

    def outer(t, carry):
        for b in range(NBUF):
            step(t * NBUF + b, b)
        return carry

    lax.fori_loop(0, NCH // NBUF, outer, 0)
    step(jnp.int32(NCH - 1), (NCH - 1) % NBUF)

    gl = NCH - 1
    last_out = out_hbm.at[pl.ds(base + gl * CH, CH)]
    pltpu.make_async_copy(bufs[gl % NBUF], last_out, wsems[gl % NBUF]).wait()


@functools.partial(jax.jit, donate_argnums=())
def _run(table, idx, pos):
    mesh = plsc.VectorSubcoreMesh(core_axis_name="c", subcore_axis_name="s")
    f = pl.kernel(
        _body,
        out_type=jax.ShapeDtypeStruct((ROWS, DIM), jnp.float32),
        mesh=mesh,
        scratch_types=[
            pltpu.VMEM((NCH, CH), jnp.int32),
            pltpu.VMEM((SEQ, DIM), jnp.float32),
            pltpu.VMEM((CH, DIM), jnp.float32),
            pltpu.VMEM((CH, DIM), jnp.float32),
            pltpu.VMEM((CH, DIM), jnp.float32),
            pltpu.SemaphoreType.DMA,
            pltpu.SemaphoreType.DMA,
            pltpu.SemaphoreType.DMA,
            pltpu.SemaphoreType.DMA,
            pltpu.SemaphoreType.DMA,
            pltpu.SemaphoreType.DMA,
        ],
    )
    return f(table, idx, pos)


def kernel(input, embed_table, pos_table):
    idx = jnp.reshape(input.astype(jnp.int32), (NW, NCH, CH))
    out = _run(embed_table, idx, pos_table)
    return jnp.reshape(out, (BATCH, SEQ, DIM))
